# Initial kernel scaffold; baseline (speedup 1.0000x reference)
import jax
import jax.numpy as jnp
from jax.experimental import pallas as pl


def kernel(ids, emb, w1, b1, w2, b2):
    raise NotImplementedError("write your pallas kernel here")


# speedup vs baseline: 6.5171x; 6.5171x over previous
import functools

import jax
import jax.numpy as jnp
from jax.experimental import pallas as pl
from jax.experimental.pallas import tpu as pltpu

_TB = 128
_GRP = 8


def _dan_kernel(ids_smem, e3_ref, w1_ref, b1_ref, w2_ref, b2_ref,
                out_ref, mean_ref, *, seq_len, tb):
    gi = pl.program_id(0)

    def body(g, carry):
        base = (gi * tb + g * _GRP) * seq_len
        rows = []
        for r in range(_GRP):
            rowbase = base + r * seq_len
            acc = e3_ref[pl.ds(ids_smem[rowbase], 1), 0, :]
            for s in range(1, seq_len):
                acc = acc + e3_ref[pl.ds(ids_smem[rowbase + s], 1), 0, :]
            rows.append(acc)
        blk = jnp.concatenate(rows, axis=0)
        start = pl.multiple_of(g * _GRP, _GRP)
        mean_ref[pl.ds(start, _GRP), :] = blk
        return carry

    jax.lax.fori_loop(0, tb // _GRP, body, 0)

    mean = mean_ref[...] * (1.0 / seq_len)
    h = jnp.dot(mean, w1_ref[...],
                preferred_element_type=jnp.float32) + b1_ref[...]
    h = jnp.maximum(h, 0.0)
    logits = jnp.dot(h, w2_ref[...],
                     preferred_element_type=jnp.float32) + b2_ref[...]
    m = jnp.max(logits, axis=1, keepdims=True)
    shifted = logits - m
    lse = jnp.log(jnp.sum(jnp.exp(shifted), axis=1, keepdims=True))
    out_ref[...] = shifted - lse


def kernel(ids, emb, w1, b1, w2, b2):
    B, S = ids.shape
    V, D = emb.shape
    H = w1.shape[1]
    O = w2.shape[1]

    nb = pl.cdiv(B, _TB)
    Bp = nb * _TB
    ids_p = ids
    if Bp != B:
        ids_p = jnp.zeros((Bp, S), jnp.int32).at[:B, :].set(ids)
    ids_flat = ids_p.reshape(Bp * S)

    e3 = emb.reshape(V, 1, D)

    out = pl.pallas_call(
        functools.partial(_dan_kernel, seq_len=S, tb=_TB),
        out_shape=jax.ShapeDtypeStruct((Bp, O), jnp.float32),
        grid=(nb,),
        in_specs=[
            pl.BlockSpec(memory_space=pltpu.SMEM),
            pl.BlockSpec((V, 1, D), lambda i: (0, 0, 0)),
            pl.BlockSpec((D, H), lambda i: (0, 0)),
            pl.BlockSpec((1, H), lambda i: (0, 0)),
            pl.BlockSpec((H, O), lambda i: (0, 0)),
            pl.BlockSpec((1, O), lambda i: (0, 0)),
        ],
        out_specs=pl.BlockSpec((_TB, O), lambda i: (i, 0)),
        scratch_shapes=[pltpu.VMEM((_TB, D), jnp.float32)],
        compiler_params=pltpu.CompilerParams(
            dimension_semantics=("parallel",)),
    )(ids_flat, e3, w1, b1, w2, b2)

    return out[:B, :]


# speedup vs baseline: 8.5956x; 1.3189x over previous
import functools

import jax
import jax.numpy as jnp
from jax.experimental import pallas as pl
from jax.experimental.pallas import tpu as pltpu

_TB = 128
_RPB = 2


def _dan_kernel(ids_smem, e3_ref, w1_ref, b1_ref, w2_ref, b2_ref,
                out_ref, mean_ref, *, seq_len, tb):
    gi = pl.program_id(0)

    def body(g, carry):
        row0 = g * _RPB
        base = (gi * tb + row0) * seq_len
        for r in range(_RPB):
            rowbase = base + r * seq_len
            acc = e3_ref[pl.ds(ids_smem[rowbase], 1), 0, :]
            for s in range(1, seq_len):
                acc = acc + e3_ref[pl.ds(ids_smem[rowbase + s], 1), 0, :]
            mean_ref[row0 + r, 0, :] = acc[0, :]
        return carry

    jax.lax.fori_loop(0, tb // _RPB, body, 0)

    mean = mean_ref[...].reshape(tb, mean_ref.shape[2]) * (1.0 / seq_len)
    h = jnp.dot(mean, w1_ref[...],
                preferred_element_type=jnp.float32) + b1_ref[...]
    h = jnp.maximum(h, 0.0)
    logits = jnp.dot(h, w2_ref[...],
                     preferred_element_type=jnp.float32) + b2_ref[...]
    m = jnp.max(logits, axis=1, keepdims=True)
    shifted = logits - m
    lse = jnp.log(jnp.sum(jnp.exp(shifted), axis=1, keepdims=True))
    out_ref[...] = shifted - lse


def kernel(ids, emb, w1, b1, w2, b2):
    B, S = ids.shape
    V, D = emb.shape
    H = w1.shape[1]
    O = w2.shape[1]

    nb = pl.cdiv(B, _TB)
    Bp = nb * _TB
    ids_p = ids
    if Bp != B:
        ids_p = jnp.zeros((Bp, S), jnp.int32).at[:B, :].set(ids)
    ids_flat = ids_p.reshape(Bp * S)

    e3 = emb.reshape(V, 1, D)

    out = pl.pallas_call(
        functools.partial(_dan_kernel, seq_len=S, tb=_TB),
        out_shape=jax.ShapeDtypeStruct((Bp, O), jnp.float32),
        grid=(nb,),
        in_specs=[
            pl.BlockSpec(memory_space=pltpu.SMEM),
            pl.BlockSpec((V, 1, D), lambda i: (0, 0, 0)),
            pl.BlockSpec((D, H), lambda i: (0, 0)),
            pl.BlockSpec((1, H), lambda i: (0, 0)),
            pl.BlockSpec((H, O), lambda i: (0, 0)),
            pl.BlockSpec((1, O), lambda i: (0, 0)),
        ],
        out_specs=pl.BlockSpec((_TB, O), lambda i: (i, 0)),
        scratch_shapes=[pltpu.VMEM((_TB, 1, D), jnp.float32)],
        compiler_params=pltpu.CompilerParams(
            dimension_semantics=("arbitrary",)),
    )(ids_flat, e3, w1, b1, w2, b2)

    return out[:B, :]
